# baseline (speedup 1.0000x reference)
import jax
import jax.numpy as jnp
from jax.experimental import pallas as pl
from jax.experimental.pallas import tpu as pltpu

HIDDEN = 1024
FMIN = 0.1
FMAX = 0.5

B = 128
F = 32768
BK = 4096
NSTEP = 4


def _block_argmax(blk, col0):
    bmax = jnp.max(blk, axis=1, keepdims=True)
    iota = jax.lax.broadcasted_iota(jnp.int32, (B, BK), 1)
    cand = jnp.where(blk == bmax, iota, F)
    bidx = jnp.min(cand, axis=1, keepdims=True) + col0
    return bmax, bidx


def _merge(m1, i1, m2, i2):
    take2 = (m2 > m1) | ((m2 == m1) & (i2 < i1))
    return jnp.where(take2, m2, m1), jnp.where(take2, i2, i1)


def _psd_peak_body(xa_ref, xb_ref, out_ref, rmax, ridx):
    k = pl.program_id(0)
    bmax_a, bidx_a = _block_argmax(xa_ref[...], k * BK)
    bmax_b, bidx_b = _block_argmax(xb_ref[...], (k + NSTEP) * BK)
    bmax, bidx = _merge(bmax_a, bidx_a, bmax_b, bidx_b)

    @pl.when(k == 0)
    def _():
        rmax[...] = bmax
        ridx[...] = bidx

    @pl.when(k > 0)
    def _():
        m, i = _merge(rmax[...], ridx[...], bmax, bidx)
        rmax[...] = m
        ridx[...] = i

    @pl.when(k == NSTEP - 1)
    def _():
        idxf = ridx[...].astype(jnp.float32)
        freq = FMIN + (FMAX - FMIN) * idxf / (F - 1)
        rr = freq * 60.0
        out_ref[...] = jnp.broadcast_to(rr, (B, HIDDEN))


_psd_peak = pl.pallas_call(
    _psd_peak_body,
    grid=(NSTEP,),
    in_specs=[
        pl.BlockSpec((B, BK), lambda k: (0, k)),
        pl.BlockSpec((B, BK), lambda k: (0, k + NSTEP)),
    ],
    out_specs=pl.BlockSpec((B, HIDDEN), lambda k: (0, 0)),
    out_shape=jax.ShapeDtypeStruct((B, HIDDEN), jnp.float32),
    scratch_shapes=[
        pltpu.VMEM((B, 1), jnp.float32),
        pltpu.VMEM((B, 1), jnp.int32),
    ],
)


def kernel(x):
    return _psd_peak(x, x)
